# Initial kernel scaffold; baseline (speedup 1.0000x reference)
import jax
import jax.numpy as jnp
from jax.experimental import pallas as pl


def kernel(x, weight):
    raise NotImplementedError("write your pallas kernel here")


# speedup vs baseline: 251.4390x; 251.4390x over previous
import jax
import jax.numpy as jnp
from jax.experimental import pallas as pl
from jax.experimental.pallas import tpu as pltpu

D0, D1 = 2048, 2048
K_KEEP = (D0 * D1) // 2
RB = 256
NB = D0 // RB


def _norm_body(x_ref, norm_ref):
    xx = x_ref[...]
    norm_ref[...] = jnp.sqrt(jnp.sum(xx * xx, axis=0, keepdims=True))


def _bits_body(w_ref, norm_ref, bits_ref):
    metric = jnp.abs(w_ref[...]) * norm_ref[...]
    bits_ref[...] = jax.lax.bitcast_convert_type(metric, jnp.int32)


def _select_body(bits_ref, t_ref):
    def count_ge(mid):
        def chunk(c, acc):
            blk = bits_ref[pl.ds(c * RB, RB), :]
            return acc + jnp.sum((blk >= mid).astype(jnp.int32))
        return jax.lax.fori_loop(0, NB, chunk, jnp.int32(0))

    def step(_, carry):
        lo, hi = carry
        mid = hi - (hi - lo) // 2
        feas = count_ge(mid) >= K_KEEP
        return (jnp.where(feas, mid, lo), jnp.where(feas, hi, mid - 1))

    lo, _ = jax.lax.fori_loop(0, 31, step, (jnp.int32(0), jnp.int32(2**31 - 1)))
    t_ref[0, 0] = lo


def _mask_body(w_ref, bits_ref, t_ref, out_ref):
    t = t_ref[0, 0]
    out_ref[...] = jnp.where(bits_ref[...] >= t, w_ref[...], 0.0)


@jax.jit
def kernel(x, weight):
    norm = pl.pallas_call(
        _norm_body,
        out_shape=jax.ShapeDtypeStruct((1, D1), jnp.float32),
    )(x)
    bits = pl.pallas_call(
        _bits_body,
        grid=(NB,),
        in_specs=[
            pl.BlockSpec((RB, D1), lambda i: (i, 0)),
            pl.BlockSpec((1, D1), lambda i: (0, 0)),
        ],
        out_specs=pl.BlockSpec((RB, D1), lambda i: (i, 0)),
        out_shape=jax.ShapeDtypeStruct((D0, D1), jnp.int32),
    )(weight, norm)
    t = pl.pallas_call(
        _select_body,
        out_shape=jax.ShapeDtypeStruct((1, 1), jnp.int32),
        out_specs=pl.BlockSpec(memory_space=pltpu.SMEM),
    )(bits)
    out = pl.pallas_call(
        _mask_body,
        grid=(NB,),
        in_specs=[
            pl.BlockSpec((RB, D1), lambda i: (i, 0)),
            pl.BlockSpec((RB, D1), lambda i: (i, 0)),
            pl.BlockSpec(memory_space=pltpu.SMEM),
        ],
        out_specs=pl.BlockSpec((RB, D1), lambda i: (i, 0)),
        out_shape=jax.ShapeDtypeStruct((D0, D1), jnp.float32),
    )(weight, bits, t)
    return out
